# baseline (speedup 1.0000x reference)
import functools

import jax
import jax.numpy as jnp
from jax import lax
from jax.experimental import pallas as pl
from jax.experimental.pallas import tpu as pltpu
from jax.experimental.pallas import tpu_sc as plsc

N = 10000
D = 128
H = 64
E = 320000
NEG = 0.2

NC = 2
NS = 16
NW = NC * NS
SB = 128
NB = 4
CH = SB * NB
CPW = 21
EPW = CH * CPW
EPAD = NW * EPW
NA = 10240
RPS = NA // NS


def _sc_edge_kernel(asv, adv, g, idx):
    mesh = plsc.VectorSubcoreMesh(core_axis_name="c", subcore_axis_name="s")
    buf = [
        pltpu.VMEM((2 * NB, SB), jnp.int32),
        pltpu.VMEM((NB, SB), jnp.int32),
        pltpu.VMEM((NB, SB), jnp.float32),
        pltpu.VMEM((NB, SB), jnp.float32),
        pltpu.VMEM((NB, SB), jnp.float32),
        pltpu.VMEM((NB, SB, H), jnp.float32),
        pltpu.SemaphoreType.DMA,
        pltpu.SemaphoreType.DMA,
        pltpu.SemaphoreType.DMA,
    ]
    kern = pl.kernel(
        _edge_body,
        out_type=[jax.ShapeDtypeStruct((NC, NA, H), jnp.float32),
                  jax.ShapeDtypeStruct((NC, NA), jnp.float32)],
        mesh=mesh,
        scratch_types=buf + buf + [
            pltpu.VMEM_SHARED((NA, H), jnp.float32),
            pltpu.VMEM_SHARED((NA,), jnp.float32),
        ],
        compiler_params=pltpu.CompilerParams(use_tc_tiling_on_sc=False),
    )
    return kern(asv, adv, g, idx)


def _edge_body(asv, adv, g, idx, acc_out, den_out, *refs):
    bufs = (refs[0:9], refs[9:18])
    acc_s, den_s = refs[18], refs[19]
    cid = lax.axis_index("c")
    sid = lax.axis_index("s")
    wid = cid * NS + sid

    def gath_descs(bp):
        idxv, _, asrcv, adstv, _, hrows, semg, _, _ = bp
        ds_ = []
        for b in range(NB):
            ds_.append(pltpu.make_async_copy(asv.at[idxv.at[b]],
                                             asrcv.at[b], semg))
            ds_.append(pltpu.make_async_copy(adv.at[idxv.at[NB + b]],
                                             adstv.at[b], semg))
            ds_.append(pltpu.make_async_copy(g.at[idxv.at[b]],
                                             hrows.at[b], semg))
        return ds_

    def scat_descs(bp):
        _, dscat, _, _, wv, hrows, _, _, sems = bp
        ds_ = []
        for b in range(NB):
            ds_.append(pltpu.make_async_copy(wv.at[b], den_s.at[dscat.at[b]],
                                             sems))
            ds_.append(pltpu.make_async_copy(hrows.at[b],
                                             acc_s.at[dscat.at[b]], sems))
        return ds_

    def compute(bp, i, first, last):
        idxv, dscat, asrcv, adstv, wv, hrows, semg, semi, sems = bp
        nxt = bufs[1] if bp is bufs[0] else bufs[0]
        for d in gath_descs(bp):
            d.wait()
        for b in range(NB):
            for t in range(SB // 16):
                dscat[b, pl.ds(t * 16, 16)] = idxv[NB + b, pl.ds(t * 16, 16)]
        if not last:
            pltpu.make_async_copy(idx.at[wid, i + 1], nxt[0], nxt[7]).wait()
        if not first:
            for d in scat_descs(nxt):
                d.wait()
        if not last:
            for d in gath_descs(nxt):
                d.start()
            pltpu.make_async_copy(idx.at[wid, i + 2], idxv, semi).start()
        for b in range(NB):
            @pl.loop(0, SB // 16)
            def _w(t, b=b):
                a = asrcv[b, pl.ds(t * 16, 16)] + adstv[b, pl.ds(t * 16, 16)]
                a = jnp.maximum(a, NEG * a)
                wv[b, pl.ds(t * 16, 16)] = jnp.exp(a)

            @pl.loop(0, SB // 16)
            def _scale(t, b=b):
                wchunk = wv[b, pl.ds(t * 16, 16)]
                for k in range(16):
                    ws = wchunk[k]
                    r = t * 16 + k
                    for q in range(H // 16):
                        hrows[b, r, pl.ds(q * 16, 16)] = (
                            hrows[b, r, pl.ds(q * 16, 16)] * ws)
        for d in scat_descs(bp):
            d.start(add=True)

    h0 = bufs[0][5]
    w0 = bufs[0][4]

    @pl.loop(0, SB)
    def _z(r):
        for q in range(H // 16):
            h0[0, r, pl.ds(q * 16, 16)] = jnp.zeros((16,), jnp.float32)
    for t in range(SB // 16):
        w0[0, pl.ds(t * 16, 16)] = jnp.zeros((16,), jnp.float32)

    for j in range(RPS // SB):
        pltpu.sync_copy(h0.at[0], acc_s.at[pl.ds(sid * RPS + j * SB, SB)])
        pltpu.sync_copy(w0.at[0], den_s.at[pl.ds(sid * RPS + j * SB, SB)])
    plsc.subcore_barrier()

    pltpu.sync_copy(idx.at[wid, 0], bufs[0][0])
    for d in gath_descs(bufs[0]):
        d.start()
    pltpu.async_copy(idx.at[wid, 1], bufs[1][0], bufs[1][7])
    compute(bufs[0], 0, first=True, last=False)

    pairs = (CPW - 2) // 2

    @pl.loop(0, pairs)
    def _pair(j):
        compute(bufs[1], 2 * j + 1, first=False, last=False)
        compute(bufs[0], 2 * j + 2, first=False, last=False)

    for i in range(2 * pairs + 1, CPW):
        compute(bufs[i % 2], i, first=False, last=(i == CPW - 1))
    for d in scat_descs(bufs[(CPW - 1) % 2]):
        d.wait()
    pltpu.make_async_copy(idx.at[wid, CPW], bufs[CPW % 2][0],
                          bufs[CPW % 2][7]).wait()

    plsc.subcore_barrier()
    pltpu.sync_copy(acc_s.at[pl.ds(sid * RPS, RPS)],
                    acc_out.at[cid, pl.ds(sid * RPS, RPS)])
    pltpu.sync_copy(den_s.at[pl.ds(sid * RPS, RPS)],
                    den_out.at[cid, pl.ds(sid * RPS, RPS)])


def _tc0_body(x_ref, wn_ref, bn_ref, wg_ref, aw_ref, g_ref, av_ref):
    h = jnp.dot(x_ref[...], wn_ref[...],
                preferred_element_type=jnp.float32) + bn_ref[...]
    gm = jnp.dot(h, wg_ref[...], preferred_element_type=jnp.float32)
    g_ref[...] = gm
    av_ref[...] = jnp.dot(gm, aw_ref[...], preferred_element_type=jnp.float32)


def _tc0(x, Wn, bn, Wg, aw):
    return pl.pallas_call(
        _tc0_body,
        out_shape=[jax.ShapeDtypeStruct((N, H), jnp.float32),
                   jax.ShapeDtypeStruct((N, 2), jnp.float32)],
    )(x, Wn, bn.reshape(1, H), Wg, aw)


def _tcmid_body(a0_ref, a1_ref, d0_ref, d1_ref, b_ref, wg_ref, aw_ref,
                g_ref, av_ref):
    den = d0_ref[...] + d1_ref[...] + 1e-16
    h = (a0_ref[...] + a1_ref[...]) / den + b_ref[...]
    h = jnp.maximum(h, 0.0)
    gm = jnp.dot(h, wg_ref[...], preferred_element_type=jnp.float32)
    g_ref[...] = gm
    av_ref[...] = jnp.dot(gm, aw_ref[...], preferred_element_type=jnp.float32)


def _tcmid(a0, a1, d0, d1, b, Wg, aw):
    return pl.pallas_call(
        _tcmid_body,
        out_shape=[jax.ShapeDtypeStruct((N, H), jnp.float32),
                   jax.ShapeDtypeStruct((N, 2), jnp.float32)],
    )(a0, a1, d0, d1, b, Wg, aw)


def _tcfin_body(a0_ref, a1_ref, d0_ref, d1_ref, b_ref,
                wp1a_ref, wp1b_ref, bp1_ref, wp2_ref, bp2_ref,
                wc1_ref, bc1_ref, wc2_ref, bc2_ref,
                wt1_ref, bt1_ref, wt2_ref, bt2_ref,
                sc_ref, ty_ref, ge_ref):
    den = d0_ref[...] + d1_ref[...] + 1e-16
    h = (a0_ref[...] + a1_ref[...]) / den + b_ref[...]
    gmean = jnp.sum(h, axis=0, keepdims=True) * (1.0 / N)
    gmax = jnp.max(h, axis=0, keepdims=True)
    pre = (jnp.dot(gmean, wp1a_ref[...], preferred_element_type=jnp.float32)
           + jnp.dot(gmax, wp1b_ref[...], preferred_element_type=jnp.float32)
           + bp1_ref[...])
    ge = jnp.dot(jnp.maximum(pre, 0.0), wp2_ref[...],
                 preferred_element_type=jnp.float32) + bp2_ref[...]
    ge_ref[...] = ge
    c = jnp.dot(jnp.maximum(
        jnp.dot(ge, wc1_ref[...], preferred_element_type=jnp.float32)
        + bc1_ref[...], 0.0), wc2_ref[...],
        preferred_element_type=jnp.float32) + bc2_ref[...]
    sc_ref[...] = 1.0 / (1.0 + jnp.exp(-c))
    ty_ref[...] = jnp.dot(jnp.maximum(
        jnp.dot(ge, wt1_ref[...], preferred_element_type=jnp.float32)
        + bt1_ref[...], 0.0), wt2_ref[...],
        preferred_element_type=jnp.float32) + bt2_ref[...]


def _tcfin(a0, a1, d0, d1, b, Wp1, bp1, Wp2, bp2,
           Wc1, bc1, Wc2, bc2, Wt1, bt1, Wt2, bt2):
    return pl.pallas_call(
        _tcfin_body,
        out_shape=[jax.ShapeDtypeStruct((1, 1), jnp.float32),
                   jax.ShapeDtypeStruct((1, 6), jnp.float32),
                   jax.ShapeDtypeStruct((1, H // 2), jnp.float32)],
    )(a0, a1, d0, d1, b,
      Wp1[:H], Wp1[H:], bp1.reshape(1, H), Wp2, bp2.reshape(1, H // 2),
      Wc1, bc1.reshape(1, H // 4), Wc2, bc2.reshape(1, 1),
      Wt1, bt1.reshape(1, H // 4), Wt2, bt2.reshape(1, 6))


def _layer_edges(edge_index):
    loop = jnp.arange(N, dtype=jnp.int32)
    pad = EPAD - (E + N)
    srcs = jnp.concatenate([edge_index[0], loop,
                            jnp.zeros((pad,), jnp.int32)])
    dsts = jnp.concatenate([edge_index[1], loop,
                            jnp.full((pad,), N, jnp.int32)])
    s4 = srcs.reshape(NW, CPW, NB, SB)
    d4 = dsts.reshape(NW, CPW, NB, SB)
    idx = jnp.concatenate([s4, d4], axis=2)
    dummy = jnp.zeros((NW, 2, 2 * NB, SB), jnp.int32)
    return jnp.concatenate([idx, dummy], axis=1)


def _sc_layer(g, av, idx):
    asv = jnp.concatenate([av[:, 0], jnp.zeros((NA - N,), jnp.float32)])
    adv = jnp.concatenate([av[:, 1], jnp.zeros((NA - N,), jnp.float32)])
    acc, den = _sc_edge_kernel(asv, adv, g, idx)
    a0 = acc[0, :N]
    a1 = acc[1, :N]
    d0 = den[0, :N].reshape(N, 1)
    d1 = den[1, :N].reshape(N, 1)
    return a0, a1, d0, d1


def kernel(x, edge_index, edge_attr, Wn, bn, Wg0, as0, ad0, bg0,
           Wg1, as1, ad1, bg1, Wg2, as2, ad2, bg2,
           Wp1, bp1, Wp2, bp2, Wc1, bc1, Wc2, bc2, Wt1, bt1, Wt2, bt2):
    idx = _layer_edges(edge_index)
    aw0 = jnp.stack([as0[0, 0], ad0[0, 0]], axis=1)
    aw1 = jnp.stack([as1[0, 0], ad1[0, 0]], axis=1)
    aw2 = jnp.stack([as2[0, 0], ad2[0, 0]], axis=1)

    g0, av0 = _tc0(x, Wn, bn, Wg0, aw0)
    a0, a1, d0, d1 = _sc_layer(g0, av0, idx)

    g1, av1 = _tcmid(a0, a1, d0, d1, bg0.reshape(1, H), Wg1, aw1)
    a0, a1, d0, d1 = _sc_layer(g1, av1, idx)

    g2, av2 = _tcmid(a0, a1, d0, d1, bg1.reshape(1, H), Wg2, aw2)
    a0, a1, d0, d1 = _sc_layer(g2, av2, idx)

    scores, types, ge = _tcfin(a0, a1, d0, d1, bg2.reshape(1, H),
                               Wp1, bp1, Wp2, bp2, Wc1, bc1, Wc2, bc2,
                               Wt1, bt1, Wt2, bt2)
    return (scores, types, ge)


# speedup vs baseline: 1.6080x; 1.6080x over previous
import jax
import jax.numpy as jnp
from jax import lax
from jax.experimental import pallas as pl
from jax.experimental.pallas import tpu as pltpu
from jax.experimental.pallas import tpu_sc as plsc

N = 10000
D = 128
H = 64
E = 320000
NEG = 0.2

NC = 2
NS = 16
NW = NC * NS
SB = 128
NB = 9
CH = SB * NB
CPW = 9
EPW = CH * CPW
EPAD = NW * EPW
NA = 10240
RPS = NA // NS


def _sc_edge_kernel(asv, adv, g, idx):
    mesh = plsc.VectorSubcoreMesh(core_axis_name="c", subcore_axis_name="s")
    kern = pl.kernel(
        _edge_body,
        out_type=[jax.ShapeDtypeStruct((NC, NA, H), jnp.float32),
                  jax.ShapeDtypeStruct((NC, NA), jnp.float32)],
        mesh=mesh,
        scratch_types=[
            pltpu.VMEM((2 * NB, SB), jnp.int32),
            pltpu.VMEM((NB, SB), jnp.float32),
            pltpu.VMEM((NB, SB), jnp.float32),
            pltpu.VMEM((NB, SB), jnp.float32),
            pltpu.VMEM((NB, SB, H), jnp.float32),
            pltpu.VMEM_SHARED((NA, H), jnp.float32),
            pltpu.VMEM_SHARED((NA,), jnp.float32),
            pltpu.SemaphoreType.DMA,
            pltpu.SemaphoreType.DMA,
        ],
        compiler_params=pltpu.CompilerParams(use_tc_tiling_on_sc=False),
    )
    return kern(asv, adv, g, idx)


def _edge_body(asv, adv, g, idx, acc_out, den_out,
               idxv, asrcv, adstv, wv, hrows, acc_s, den_s, semg, sems):
    cid = lax.axis_index("c")
    sid = lax.axis_index("s")
    wid = cid * NS + sid

    @pl.loop(0, SB)
    def _z(r):
        for q in range(H // 16):
            hrows[0, r, pl.ds(q * 16, 16)] = jnp.zeros((16,), jnp.float32)
    for b in range(NB):
        for t in range(SB // 16):
            wv[b, pl.ds(t * 16, 16)] = jnp.zeros((16,), jnp.float32)

    for j in range(RPS // SB):
        pltpu.sync_copy(hrows.at[0], acc_s.at[pl.ds(sid * RPS + j * SB, SB)])
        pltpu.sync_copy(wv.at[0], den_s.at[pl.ds(sid * RPS + j * SB, SB)])
    plsc.subcore_barrier()

    @pl.loop(0, CPW)
    def _chunk(i):
        pltpu.sync_copy(idx.at[wid, i], idxv)
        gat = []
        for b in range(NB):
            gat.append(pltpu.async_copy(asv.at[idxv.at[b]], asrcv.at[b], semg))
        for b in range(NB):
            gat.append(pltpu.async_copy(adv.at[idxv.at[NB + b]],
                                        adstv.at[b], semg))
        for b in range(NB):
            gat.append(pltpu.async_copy(g.at[idxv.at[b]], hrows.at[b], semg))
        for d in gat:
            d.wait()

        for b in range(NB):
            @pl.loop(0, SB // 16)
            def _w(t, b=b):
                a = asrcv[b, pl.ds(t * 16, 16)] + adstv[b, pl.ds(t * 16, 16)]
                a = jnp.maximum(a, NEG * a)
                wv[b, pl.ds(t * 16, 16)] = jnp.exp(a)

            @pl.loop(0, SB // 16)
            def _scale(t, b=b):
                wchunk = wv[b, pl.ds(t * 16, 16)]
                for k in range(16):
                    ws = wchunk[k]
                    r = t * 16 + k
                    for q in range(H // 16):
                        hrows[b, r, pl.ds(q * 16, 16)] = (
                            hrows[b, r, pl.ds(q * 16, 16)] * ws)

        sca = []
        for b in range(NB):
            sca.append(pltpu.async_copy(wv.at[b], den_s.at[idxv.at[NB + b]],
                                        sems, add=True))
            sca.append(pltpu.async_copy(hrows.at[b], acc_s.at[idxv.at[NB + b]],
                                        sems, add=True))
        for d in sca:
            d.wait()

    plsc.subcore_barrier()
    pltpu.sync_copy(acc_s.at[pl.ds(sid * RPS, RPS)],
                    acc_out.at[cid, pl.ds(sid * RPS, RPS)])
    pltpu.sync_copy(den_s.at[pl.ds(sid * RPS, RPS)],
                    den_out.at[cid, pl.ds(sid * RPS, RPS)])


def _tc0_body(x_ref, wn_ref, bn_ref, wg_ref, aw_ref, g_ref, av_ref):
    h = jnp.dot(x_ref[...], wn_ref[...],
                preferred_element_type=jnp.float32) + bn_ref[...]
    gm = jnp.dot(h, wg_ref[...], preferred_element_type=jnp.float32)
    g_ref[...] = gm
    av_ref[...] = jnp.dot(gm, aw_ref[...], preferred_element_type=jnp.float32)


def _tc0(x, Wn, bn, Wg, aw):
    return pl.pallas_call(
        _tc0_body,
        out_shape=[jax.ShapeDtypeStruct((N, H), jnp.float32),
                   jax.ShapeDtypeStruct((N, 2), jnp.float32)],
    )(x, Wn, bn.reshape(1, H), Wg, aw)


def _tcmid_body(a0_ref, a1_ref, d0_ref, d1_ref, b_ref, wg_ref, aw_ref,
                g_ref, av_ref):
    den = d0_ref[...] + d1_ref[...] + 1e-16
    h = (a0_ref[...] + a1_ref[...]) / den + b_ref[...]
    h = jnp.maximum(h, 0.0)
    gm = jnp.dot(h, wg_ref[...], preferred_element_type=jnp.float32)
    g_ref[...] = gm
    av_ref[...] = jnp.dot(gm, aw_ref[...], preferred_element_type=jnp.float32)


def _tcmid(a0, a1, d0, d1, b, Wg, aw):
    return pl.pallas_call(
        _tcmid_body,
        out_shape=[jax.ShapeDtypeStruct((N, H), jnp.float32),
                   jax.ShapeDtypeStruct((N, 2), jnp.float32)],
    )(a0, a1, d0, d1, b, Wg, aw)


def _tcfin_body(a0_ref, a1_ref, d0_ref, d1_ref, b_ref,
                wp1a_ref, wp1b_ref, bp1_ref, wp2_ref, bp2_ref,
                wc1_ref, bc1_ref, wc2_ref, bc2_ref,
                wt1_ref, bt1_ref, wt2_ref, bt2_ref,
                sc_ref, ty_ref, ge_ref):
    den = d0_ref[...] + d1_ref[...] + 1e-16
    h = (a0_ref[...] + a1_ref[...]) / den + b_ref[...]
    gmean = jnp.sum(h, axis=0, keepdims=True) * (1.0 / N)
    gmax = jnp.max(h, axis=0, keepdims=True)
    pre = (jnp.dot(gmean, wp1a_ref[...], preferred_element_type=jnp.float32)
           + jnp.dot(gmax, wp1b_ref[...], preferred_element_type=jnp.float32)
           + bp1_ref[...])
    ge = jnp.dot(jnp.maximum(pre, 0.0), wp2_ref[...],
                 preferred_element_type=jnp.float32) + bp2_ref[...]
    ge_ref[...] = ge
    c = jnp.dot(jnp.maximum(
        jnp.dot(ge, wc1_ref[...], preferred_element_type=jnp.float32)
        + bc1_ref[...], 0.0), wc2_ref[...],
        preferred_element_type=jnp.float32) + bc2_ref[...]
    sc_ref[...] = 1.0 / (1.0 + jnp.exp(-c))
    ty_ref[...] = jnp.dot(jnp.maximum(
        jnp.dot(ge, wt1_ref[...], preferred_element_type=jnp.float32)
        + bt1_ref[...], 0.0), wt2_ref[...],
        preferred_element_type=jnp.float32) + bt2_ref[...]


def _tcfin(a0, a1, d0, d1, b, Wp1, bp1, Wp2, bp2,
           Wc1, bc1, Wc2, bc2, Wt1, bt1, Wt2, bt2):
    return pl.pallas_call(
        _tcfin_body,
        out_shape=[jax.ShapeDtypeStruct((1, 1), jnp.float32),
                   jax.ShapeDtypeStruct((1, 6), jnp.float32),
                   jax.ShapeDtypeStruct((1, H // 2), jnp.float32)],
    )(a0, a1, d0, d1, b,
      Wp1[:H], Wp1[H:], bp1.reshape(1, H), Wp2, bp2.reshape(1, H // 2),
      Wc1, bc1.reshape(1, H // 4), Wc2, bc2.reshape(1, 1),
      Wt1, bt1.reshape(1, H // 4), Wt2, bt2.reshape(1, 6))


def _layer_edges(edge_index):
    loop = jnp.arange(N, dtype=jnp.int32)
    pad = EPAD - (E + N)
    srcs = jnp.concatenate([edge_index[0], loop,
                            jnp.zeros((pad,), jnp.int32)])
    dsts = jnp.concatenate([edge_index[1], loop,
                            jnp.full((pad,), N, jnp.int32)])
    s4 = srcs.reshape(NW, CPW, NB, SB)
    d4 = dsts.reshape(NW, CPW, NB, SB)
    return jnp.concatenate([s4, d4], axis=2)


def _sc_layer(g, av, idx):
    asv = jnp.concatenate([av[:, 0], jnp.zeros((NA - N,), jnp.float32)])
    adv = jnp.concatenate([av[:, 1], jnp.zeros((NA - N,), jnp.float32)])
    acc, den = _sc_edge_kernel(asv, adv, g, idx)
    a0 = acc[0, :N]
    a1 = acc[1, :N]
    d0 = den[0, :N].reshape(N, 1)
    d1 = den[1, :N].reshape(N, 1)
    return a0, a1, d0, d1


def kernel(x, edge_index, edge_attr, Wn, bn, Wg0, as0, ad0, bg0,
           Wg1, as1, ad1, bg1, Wg2, as2, ad2, bg2,
           Wp1, bp1, Wp2, bp2, Wc1, bc1, Wc2, bc2, Wt1, bt1, Wt2, bt2):
    idx = _layer_edges(edge_index)
    aw0 = jnp.stack([as0[0, 0], ad0[0, 0]], axis=1)
    aw1 = jnp.stack([as1[0, 0], ad1[0, 0]], axis=1)
    aw2 = jnp.stack([as2[0, 0], ad2[0, 0]], axis=1)

    g0, av0 = _tc0(x, Wn, bn, Wg0, aw0)
    a0, a1, d0, d1 = _sc_layer(g0, av0, idx)

    g1, av1 = _tcmid(a0, a1, d0, d1, bg0.reshape(1, H), Wg1, aw1)
    a0, a1, d0, d1 = _sc_layer(g1, av1, idx)

    g2, av2 = _tcmid(a0, a1, d0, d1, bg1.reshape(1, H), Wg2, aw2)
    a0, a1, d0, d1 = _sc_layer(g2, av2, idx)

    scores, types, ge = _tcfin(a0, a1, d0, d1, bg2.reshape(1, H),
                               Wp1, bp1, Wp2, bp2, Wc1, bc1, Wc2, bc2,
                               Wt1, bt1, Wt2, bt2)
    return (scores, types, ge)
